# baseline (speedup 1.0000x reference)
import functools

import jax
import jax.numpy as jnp
from jax import lax
from jax.experimental import pallas as pl
from jax.experimental.pallas import tpu as pltpu
from jax.experimental.pallas import tpu_sc as plsc

N = 10000
E = 320000
C = 128
SW = 16
BN = 1000
NB = N // BN
NWORK = 32
EPW = E // NWORK
CHUNK = 80
NCHUNK = EPW // CHUNK
GROUPS = CHUNK // 16
ROWS_PER_TILE = N // 16


def _prep_body(x_ref, w_ref, a2_ref, h_ref, side_ref, amax_ref):
    i = pl.program_id(0)
    h = lax.dot_general(x_ref[...], w_ref[...], (((1,), (0,)), ((), ())),
                        preferred_element_type=jnp.float32,
                        precision=lax.Precision.DEFAULT)
    al = lax.dot_general(h, a2_ref[...], (((1,), (0,)), ((), ())),
                         preferred_element_type=jnp.float32,
                         precision=lax.Precision.HIGHEST)
    h_ref[...] = h
    side_ref[...] = jnp.concatenate(
        [al, jnp.zeros((BN, SW - 2), jnp.float32)], axis=1)
    bm = jnp.full((1, 128), jnp.max(al[:, 0]), jnp.float32)

    @pl.when(i == 0)
    def _():
        amax_ref[...] = bm

    @pl.when(i > 0)
    def _():
        amax_ref[...] = jnp.maximum(amax_ref[...], bm)


def _fin_body(p0_ref, p1_ref, d0_ref, d1_ref, b_ref, o_ref):
    num = p0_ref[0] + p1_ref[0]
    den = d0_ref[0, :, 0:1] + d1_ref[0, :, 0:1] + 1e-9
    o = num / den + b_ref[...]
    o_ref[...] = jnp.maximum(o, 0.3 * o)


def _sc_body(h_hbm, side_hbm, amax_hbm, ei_hbm,
             outh_hbm, outd_hbm,
             exi_v, scat_v, rows_v, dens_v, sbuf_v, a_tab,
             acc_h, acc_d, gsem, ssem, isem):
    c = lax.axis_index("c")
    s = lax.axis_index("s")
    wid = c * 16 + s

    pltpu.sync_copy(amax_hbm, a_tab)
    ebase = wid * EPW

    @pl.loop(0, CHUNK)
    def _(i):
        for j in range(C // 16):
            rows_v[i, pl.ds(j * 16, 16)] = jnp.zeros((16,), jnp.float32)
        dens_v[i, :] = jnp.zeros((16,), jnp.float32)

    stripe = s * ROWS_PER_TILE

    @pl.loop(0, ROWS_PER_TILE // CHUNK)
    def _(k):
        pltpu.sync_copy(rows_v.at[pl.ds(0, CHUNK)],
                        acc_h.at[pl.ds(stripe + k * CHUNK, CHUNK)])
        pltpu.sync_copy(dens_v.at[pl.ds(0, CHUNK)],
                        acc_d.at[pl.ds(stripe + k * CHUNK, CHUNK)])

    rem = ROWS_PER_TILE % CHUNK
    if rem:
        base = stripe + (ROWS_PER_TILE // CHUNK) * CHUNK
        pltpu.sync_copy(rows_v.at[pl.ds(0, rem)], acc_h.at[pl.ds(base, rem)])
        pltpu.sync_copy(dens_v.at[pl.ds(0, rem)], acc_d.at[pl.ds(base, rem)])

    plsc.subcore_barrier()

    a_reg = a_tab[...]
    lane = lax.iota(jnp.int32, 16)
    col0 = jnp.full((16,), 0, jnp.int32)
    col1 = jnp.full((16,), 1, jnp.int32)

    gbase = wid * (2 * EPW)
    base0 = pl.multiple_of(gbase, 8)
    pltpu.async_copy(ei_hbm.at[pl.ds(base0, 2 * CHUNK)], exi_v.at[0], isem)
    pltpu.make_async_copy(ei_hbm.at[pl.ds(base0, 2 * CHUNK)], exi_v.at[0], isem).wait()
    pltpu.async_copy(h_hbm.at[exi_v.at[0, pl.ds(0, CHUNK)]],
                     rows_v.at[pl.ds(0, CHUNK)], gsem)
    pltpu.async_copy(side_hbm.at[exi_v.at[0]], sbuf_v.at[pl.ds(0, 2 * CHUNK)], gsem)
    base1 = pl.multiple_of(gbase + 2 * CHUNK, 8)
    pltpu.async_copy(ei_hbm.at[pl.ds(base1, 2 * CHUNK)], exi_v.at[1], isem)

    def _wait_gathers(off):
        pltpu.make_async_copy(h_hbm.at[exi_v.at[0, pl.ds(0, CHUNK)]],
                              rows_v.at[pl.ds(off, CHUNK)], gsem).wait()
        pltpu.make_async_copy(side_hbm.at[exi_v.at[0]],
                              sbuf_v.at[pl.ds(2 * off, 2 * CHUNK)], gsem).wait()

    def _wait_scatters(off, sb):
        pltpu.make_async_copy(rows_v.at[pl.ds(off, CHUNK)],
                              acc_h.at[scat_v.at[sb]], ssem).wait()
        pltpu.make_async_copy(dens_v.at[pl.ds(off, CHUNK)],
                              acc_d.at[scat_v.at[sb]], ssem).wait()

    @pl.loop(0, NCHUNK)
    def _(ci):
        b = lax.rem(ci, 2)
        off = b * CHUNK
        oth = (1 - b) * CHUNK
        _wait_gathers(off)
        @pl.when(ci >= 1)
        def _():
            _wait_scatters(oth, 1 - b)

        @pl.when(ci + 1 < NCHUNK)
        def _():
            pltpu.make_async_copy(ei_hbm.at[pl.ds(base0, 2 * CHUNK)],
                                  exi_v.at[1 - b], isem).wait()
            pltpu.async_copy(h_hbm.at[exi_v.at[1 - b, pl.ds(0, CHUNK)]],
                             rows_v.at[pl.ds(oth, CHUNK)], gsem)
            pltpu.async_copy(side_hbm.at[exi_v.at[1 - b]],
                             sbuf_v.at[pl.ds(2 * oth, 2 * CHUNK)], gsem)

        @pl.loop(0, GROUPS)
        def _(g):
            didx = exi_v[b, pl.ds(CHUNK + g * 16, 16)]
            scat_v[b, pl.ds(g * 16, 16)] = didx
            as_v = plsc.load_gather(sbuf_v, [2 * off + g * 16 + lane, col0])
            p_v = plsc.load_gather(sbuf_v, [2 * off + CHUNK + g * 16 + lane, col1])
            t = as_v + p_v
            lr = jnp.maximum(t, 0.2 * t)
            t2 = a_reg + p_v
            q = jnp.maximum(t2, 0.2 * t2)
            w = jnp.exp(lr - q)
            for k in range(16):
                wk = jnp.broadcast_to(w[k], (16,))
                row = off + g * 16 + k
                dens_v[row, :] = wk
                for j in range(C // 16):
                    rows_v[row, pl.ds(j * 16, 16)] = (
                        rows_v[row, pl.ds(j * 16, 16)] * wk)

        @pl.when(ci + 2 < NCHUNK)
        def _():
            nxt = pl.multiple_of(gbase + (ci + 2) * 2 * CHUNK, 8)
            pltpu.async_copy(ei_hbm.at[pl.ds(nxt, 2 * CHUNK)], exi_v.at[b], isem)

        pltpu.async_copy(rows_v.at[pl.ds(off, CHUNK)],
                         acc_h.at[scat_v.at[b]], ssem, add=True)
        pltpu.async_copy(dens_v.at[pl.ds(off, CHUNK)],
                         acc_d.at[scat_v.at[b]], ssem, add=True)

    lb = (NCHUNK - 1) % 2
    _wait_scatters(lb * CHUNK, lb)
    plsc.subcore_barrier()
    pltpu.sync_copy(acc_h.at[pl.ds(stripe, ROWS_PER_TILE)],
                    outh_hbm.at[c].at[pl.ds(stripe, ROWS_PER_TILE)])
    pltpu.sync_copy(acc_d.at[pl.ds(stripe, ROWS_PER_TILE)],
                    outd_hbm.at[c].at[pl.ds(stripe, ROWS_PER_TILE)])


@functools.partial(
    pl.kernel,
    out_type=[jax.ShapeDtypeStruct((2, N, C), jnp.float32),
              jax.ShapeDtypeStruct((2, N, SW), jnp.float32)],
    mesh=plsc.VectorSubcoreMesh(core_axis_name="c", subcore_axis_name="s",
                                num_cores=2, num_subcores=16),
    scratch_types=[
        pltpu.VMEM((2, 2 * CHUNK), jnp.int32),
        pltpu.VMEM((2, CHUNK), jnp.int32),
        pltpu.VMEM((2 * CHUNK, C), jnp.float32),
        pltpu.VMEM((2 * CHUNK, SW), jnp.float32),
        pltpu.VMEM((4 * CHUNK, SW), jnp.float32),
        pltpu.VMEM((16,), jnp.float32),
        pltpu.VMEM_SHARED((N, C), jnp.float32),
        pltpu.VMEM_SHARED((N, SW), jnp.float32),
        pltpu.SemaphoreType.DMA,
        pltpu.SemaphoreType.DMA,
        pltpu.SemaphoreType.DMA,
    ],
    compiler_params=pltpu.CompilerParams(use_tc_tiling_on_sc=False,
                                         needs_layout_passes=False),
)
def _sc_edge_kernel(h_hbm, side_hbm, amax_hbm, ei_hbm,
                    outh_hbm, outd_hbm,
                    exi_v, scat_v, rows_v, dens_v, sbuf_v, a_tab,
                    acc_h, acc_d, gsem, ssem, isem):
    _sc_body(h_hbm, side_hbm, amax_hbm, ei_hbm,
             outh_hbm, outd_hbm,
             exi_v, scat_v, rows_v, dens_v, sbuf_v, a_tab,
             acc_h, acc_d, gsem, ssem, isem)


def kernel(x, edge_index, W, a_src, a_dst, b):
    ei32 = edge_index.astype(jnp.int32)
    ei = jnp.stack([ei32[0].reshape(-1, CHUNK),
                    ei32[1].reshape(-1, CHUNK)], axis=1).reshape(-1)
    a2 = jnp.stack([a_src, a_dst], axis=1)

    h, side, amax = pl.pallas_call(
        _prep_body,
        grid=(NB,),
        in_specs=[
            pl.BlockSpec((BN, C), lambda i: (i, 0)),
            pl.BlockSpec((C, C), lambda i: (0, 0)),
            pl.BlockSpec((C, 2), lambda i: (0, 0)),
        ],
        out_specs=[
            pl.BlockSpec((BN, C), lambda i: (i, 0)),
            pl.BlockSpec((BN, SW), lambda i: (i, 0)),
            pl.BlockSpec((1, 128), lambda i: (0, 0)),
        ],
        out_shape=[
            jax.ShapeDtypeStruct((N, C), jnp.float32),
            jax.ShapeDtypeStruct((N, SW), jnp.float32),
            jax.ShapeDtypeStruct((1, 128), jnp.float32),
        ],
    )(x, W, a2)

    amax16 = amax[0, :16]

    part_h, part_d = _sc_edge_kernel(h, side, amax16, ei)

    out = pl.pallas_call(
        _fin_body,
        grid=(NB,),
        in_specs=[
            pl.BlockSpec((1, BN, C), lambda i: (0, i, 0)),
            pl.BlockSpec((1, BN, C), lambda i: (1, i, 0)),
            pl.BlockSpec((1, BN, SW), lambda i: (0, i, 0)),
            pl.BlockSpec((1, BN, SW), lambda i: (1, i, 0)),
            pl.BlockSpec((1, C), lambda i: (0, 0)),
        ],
        out_specs=pl.BlockSpec((BN, C), lambda i: (i, 0)),
        out_shape=jax.ShapeDtypeStruct((N, C), jnp.float32),
    )(part_h, part_h, part_d, part_d, b.reshape(1, C))
    return out


# speedup vs baseline: 1.0953x; 1.0953x over previous
import functools

import jax
import jax.numpy as jnp
from jax import lax
from jax.experimental import pallas as pl
from jax.experimental.pallas import tpu as pltpu
from jax.experimental.pallas import tpu_sc as plsc

N = 10000
E = 320000
C = 128
SW = 16
BN = 1000
NB = N // BN
NWORK = 32
EPW = E // NWORK
CHUNK = 80
NCHUNK = EPW // CHUNK
GROUPS = CHUNK // 16
ROWS_PER_TILE = N // 16


def _prep_body(x_ref, w_ref, a2_ref, h_ref, side_ref, amax_ref):
    i = pl.program_id(0)
    h = lax.dot_general(x_ref[...], w_ref[...], (((1,), (0,)), ((), ())),
                        preferred_element_type=jnp.float32,
                        precision=lax.Precision.DEFAULT)
    al = lax.dot_general(h, a2_ref[...], (((1,), (0,)), ((), ())),
                         preferred_element_type=jnp.float32,
                         precision=lax.Precision.HIGHEST)
    h_ref[...] = h
    side_ref[...] = jnp.concatenate(
        [al, jnp.zeros((BN, SW - 2), jnp.float32)], axis=1)
    bm = jnp.full((1, 128), jnp.max(al[:, 0]), jnp.float32)

    @pl.when(i == 0)
    def _():
        amax_ref[...] = bm

    @pl.when(i > 0)
    def _():
        amax_ref[...] = jnp.maximum(amax_ref[...], bm)


def _fin_body(p0_ref, p1_ref, d0_ref, d1_ref, b_ref, o_ref):
    num = p0_ref[0] + p1_ref[0]
    den = d0_ref[0, :, 0:1] + d1_ref[0, :, 0:1] + 1e-9
    o = num / den + b_ref[...]
    o_ref[...] = jnp.maximum(o, 0.3 * o)


def _sc_body(h_hbm, side_hbm, amax_hbm, src_hbm, dst_hbm,
             outh_hbm, outd_hbm,
             si_v, didx_v, scat_v, rows_v, dens_v, ss_v, sd_v, a_tab,
             acc_h, acc_d, gsem, ssem, isem):
    c = lax.axis_index("c")
    s = lax.axis_index("s")
    wid = c * 16 + s

    pltpu.sync_copy(amax_hbm, a_tab)
    ebase = wid * EPW

    @pl.loop(0, CHUNK)
    def _(i):
        for j in range(C // 16):
            rows_v[i, pl.ds(j * 16, 16)] = jnp.zeros((16,), jnp.float32)
        dens_v[i, :] = jnp.zeros((16,), jnp.float32)

    stripe = s * ROWS_PER_TILE

    @pl.loop(0, ROWS_PER_TILE // CHUNK)
    def _(k):
        pltpu.sync_copy(rows_v.at[pl.ds(0, CHUNK)],
                        acc_h.at[pl.ds(stripe + k * CHUNK, CHUNK)])
        pltpu.sync_copy(dens_v.at[pl.ds(0, CHUNK)],
                        acc_d.at[pl.ds(stripe + k * CHUNK, CHUNK)])

    rem = ROWS_PER_TILE % CHUNK
    if rem:
        base = stripe + (ROWS_PER_TILE // CHUNK) * CHUNK
        pltpu.sync_copy(rows_v.at[pl.ds(0, rem)], acc_h.at[pl.ds(base, rem)])
        pltpu.sync_copy(dens_v.at[pl.ds(0, rem)], acc_d.at[pl.ds(base, rem)])

    plsc.subcore_barrier()

    a_reg = a_tab[...]
    lane = lax.iota(jnp.int32, 16)
    col0 = jnp.full((16,), 0, jnp.int32)
    col1 = jnp.full((16,), 1, jnp.int32)

    base0 = pl.multiple_of(ebase, 8)
    pltpu.async_copy(src_hbm.at[pl.ds(base0, CHUNK)], si_v.at[0], isem)
    pltpu.async_copy(dst_hbm.at[pl.ds(base0, CHUNK)], didx_v.at[0], isem)
    pltpu.make_async_copy(src_hbm.at[pl.ds(base0, CHUNK)], si_v.at[0], isem).wait()
    pltpu.make_async_copy(dst_hbm.at[pl.ds(base0, CHUNK)], didx_v.at[0], isem).wait()
    pltpu.async_copy(h_hbm.at[si_v.at[0]], rows_v.at[pl.ds(0, CHUNK)], gsem)
    pltpu.async_copy(side_hbm.at[si_v.at[0]], ss_v.at[pl.ds(0, CHUNK)], gsem)
    pltpu.async_copy(side_hbm.at[didx_v.at[0]], sd_v.at[pl.ds(0, CHUNK)], gsem)
    base1 = pl.multiple_of(ebase + CHUNK, 8)
    pltpu.async_copy(src_hbm.at[pl.ds(base1, CHUNK)], si_v.at[1], isem)
    pltpu.async_copy(dst_hbm.at[pl.ds(base1, CHUNK)], didx_v.at[1], isem)

    def _wait_gathers(off):
        pltpu.make_async_copy(h_hbm.at[si_v.at[0]],
                              rows_v.at[pl.ds(off, CHUNK)], gsem).wait()
        pltpu.make_async_copy(side_hbm.at[si_v.at[0]],
                              ss_v.at[pl.ds(off, CHUNK)], gsem).wait()
        pltpu.make_async_copy(side_hbm.at[didx_v.at[0]],
                              sd_v.at[pl.ds(off, CHUNK)], gsem).wait()

    def _wait_scatters(off, sb):
        pltpu.make_async_copy(rows_v.at[pl.ds(off, CHUNK)],
                              acc_h.at[scat_v.at[sb]], ssem).wait()
        pltpu.make_async_copy(dens_v.at[pl.ds(off, CHUNK)],
                              acc_d.at[scat_v.at[sb]], ssem).wait()

    @pl.loop(0, NCHUNK)
    def _(ci):
        b = lax.rem(ci, 2)
        off = b * CHUNK
        oth = (1 - b) * CHUNK
        _wait_gathers(off)
        @pl.when(ci >= 1)
        def _():
            _wait_scatters(oth, 1 - b)

        @pl.when(ci + 1 < NCHUNK)
        def _():
            pltpu.make_async_copy(src_hbm.at[pl.ds(base0, CHUNK)],
                                  si_v.at[1 - b], isem).wait()
            pltpu.make_async_copy(dst_hbm.at[pl.ds(base0, CHUNK)],
                                  didx_v.at[1 - b], isem).wait()
            pltpu.async_copy(h_hbm.at[si_v.at[1 - b]],
                             rows_v.at[pl.ds(oth, CHUNK)], gsem)
            pltpu.async_copy(side_hbm.at[si_v.at[1 - b]],
                             ss_v.at[pl.ds(oth, CHUNK)], gsem)
            pltpu.async_copy(side_hbm.at[didx_v.at[1 - b]],
                             sd_v.at[pl.ds(oth, CHUNK)], gsem)

        @pl.loop(0, GROUPS)
        def _(g):
            didx = didx_v[b, pl.ds(g * 16, 16)]
            scat_v[b, pl.ds(g * 16, 16)] = didx
            as_v = plsc.load_gather(ss_v, [off + g * 16 + lane, col0])
            p_v = plsc.load_gather(sd_v, [off + g * 16 + lane, col1])
            t = as_v + p_v
            lr = jnp.maximum(t, 0.2 * t)
            t2 = a_reg + p_v
            q = jnp.maximum(t2, 0.2 * t2)
            w = jnp.exp(lr - q)
            for k in range(16):
                wk = jnp.broadcast_to(w[k], (16,))
                row = off + g * 16 + k
                dens_v[row, :] = wk
                for j in range(C // 16):
                    rows_v[row, pl.ds(j * 16, 16)] = (
                        rows_v[row, pl.ds(j * 16, 16)] * wk)

        @pl.when(ci + 2 < NCHUNK)
        def _():
            nxt = pl.multiple_of(ebase + (ci + 2) * CHUNK, 8)
            pltpu.async_copy(src_hbm.at[pl.ds(nxt, CHUNK)], si_v.at[b], isem)
            pltpu.async_copy(dst_hbm.at[pl.ds(nxt, CHUNK)], didx_v.at[b], isem)

        pltpu.async_copy(rows_v.at[pl.ds(off, CHUNK)],
                         acc_h.at[scat_v.at[b]], ssem, add=True)
        pltpu.async_copy(dens_v.at[pl.ds(off, CHUNK)],
                         acc_d.at[scat_v.at[b]], ssem, add=True)

    lb = (NCHUNK - 1) % 2
    _wait_scatters(lb * CHUNK, lb)
    plsc.subcore_barrier()
    pltpu.sync_copy(acc_h.at[pl.ds(stripe, ROWS_PER_TILE)],
                    outh_hbm.at[c].at[pl.ds(stripe, ROWS_PER_TILE)])
    pltpu.sync_copy(acc_d.at[pl.ds(stripe, ROWS_PER_TILE)],
                    outd_hbm.at[c].at[pl.ds(stripe, ROWS_PER_TILE)])


@functools.partial(
    pl.kernel,
    out_type=[jax.ShapeDtypeStruct((2, N, C), jnp.float32),
              jax.ShapeDtypeStruct((2, N, SW), jnp.float32)],
    mesh=plsc.VectorSubcoreMesh(core_axis_name="c", subcore_axis_name="s",
                                num_cores=2, num_subcores=16),
    scratch_types=[
        pltpu.VMEM((2, CHUNK), jnp.int32),
        pltpu.VMEM((2, CHUNK), jnp.int32),
        pltpu.VMEM((2, CHUNK), jnp.int32),
        pltpu.VMEM((2 * CHUNK, C), jnp.float32),
        pltpu.VMEM((2 * CHUNK, SW), jnp.float32),
        pltpu.VMEM((2 * CHUNK, SW), jnp.float32),
        pltpu.VMEM((2 * CHUNK, SW), jnp.float32),
        pltpu.VMEM((16,), jnp.float32),
        pltpu.VMEM_SHARED((N, C), jnp.float32),
        pltpu.VMEM_SHARED((N, SW), jnp.float32),
        pltpu.SemaphoreType.DMA,
        pltpu.SemaphoreType.DMA,
        pltpu.SemaphoreType.DMA,
    ],
    compiler_params=pltpu.CompilerParams(use_tc_tiling_on_sc=False,
                                         needs_layout_passes=False),
)
def _sc_edge_kernel(h_hbm, side_hbm, amax_hbm, src_hbm, dst_hbm,
                    outh_hbm, outd_hbm,
                    si_v, didx_v, scat_v, rows_v, dens_v, ss_v, sd_v, a_tab,
                    acc_h, acc_d, gsem, ssem, isem):
    _sc_body(h_hbm, side_hbm, amax_hbm, src_hbm, dst_hbm,
             outh_hbm, outd_hbm,
             si_v, didx_v, scat_v, rows_v, dens_v, ss_v, sd_v, a_tab,
             acc_h, acc_d, gsem, ssem, isem)


def kernel(x, edge_index, W, a_src, a_dst, b):
    src = edge_index[0].astype(jnp.int32)
    dst = edge_index[1].astype(jnp.int32)
    a2 = jnp.stack([a_src, a_dst], axis=1)

    h, side, amax = pl.pallas_call(
        _prep_body,
        grid=(NB,),
        in_specs=[
            pl.BlockSpec((BN, C), lambda i: (i, 0)),
            pl.BlockSpec((C, C), lambda i: (0, 0)),
            pl.BlockSpec((C, 2), lambda i: (0, 0)),
        ],
        out_specs=[
            pl.BlockSpec((BN, C), lambda i: (i, 0)),
            pl.BlockSpec((BN, SW), lambda i: (i, 0)),
            pl.BlockSpec((1, 128), lambda i: (0, 0)),
        ],
        out_shape=[
            jax.ShapeDtypeStruct((N, C), jnp.float32),
            jax.ShapeDtypeStruct((N, SW), jnp.float32),
            jax.ShapeDtypeStruct((1, 128), jnp.float32),
        ],
    )(x, W, a2)

    amax16 = amax[0, :16]

    part_h, part_d = _sc_edge_kernel(h, side, amax16, src, dst)

    out = pl.pallas_call(
        _fin_body,
        grid=(NB,),
        in_specs=[
            pl.BlockSpec((1, BN, C), lambda i: (0, i, 0)),
            pl.BlockSpec((1, BN, C), lambda i: (1, i, 0)),
            pl.BlockSpec((1, BN, SW), lambda i: (0, i, 0)),
            pl.BlockSpec((1, BN, SW), lambda i: (1, i, 0)),
            pl.BlockSpec((1, C), lambda i: (0, 0)),
        ],
        out_specs=pl.BlockSpec((BN, C), lambda i: (i, 0)),
        out_shape=jax.ShapeDtypeStruct((N, C), jnp.float32),
    )(part_h, part_h, part_d, part_d, b.reshape(1, C))
    return out
